# baseline (speedup 1.0000x reference)
import functools

import jax
import jax.numpy as jnp
from jax import lax
from jax.experimental import pallas as pl
from jax.experimental.pallas import tpu as pltpu
from jax.experimental.pallas import tpu_sc as plsc

_THRESH = (0.5333333611488342, 0.6000000238418579, 0.6666666865348816,
           0.7333333492279053, 0.8000000715255737, 0.8666667342185974,
           0.9333333969116211)

_ROWS = 8192
_COLS = 2048
_BLOCK_ROWS = 256
_GRID = _ROWS // _BLOCK_ROWS


def _ece_body(x_ref, m_ref, t_ref, out_ref):
    @pl.when(pl.program_id(0) == 0)
    def _init():
        for k in range(24):
            out_ref[k] = 0.0

    x = x_ref[...]
    mf = m_ref[...].astype(jnp.float32)
    t = t_ref[...]
    conf = 0.5 * jnp.tanh(0.5 * jnp.abs(x)) + 0.5
    acc = jnp.where(x > 0, t, 1.0 - t) * mf
    confm = conf * mf
    out_ref[0] += jnp.sum(mf)
    out_ref[1] += jnp.sum(confm)
    out_ref[2] += jnp.sum(acc)
    for k, th in enumerate(_THRESH):
        g = conf > th
        base = 3 * (k + 1)
        out_ref[base + 0] += jnp.sum(jnp.where(g, mf, 0.0))
        out_ref[base + 1] += jnp.sum(jnp.where(g, confm, 0.0))
        out_ref[base + 2] += jnp.sum(jnp.where(g, acc, 0.0))


def _partial_sums(logits, mask, targets, interpret=False):
    blk = pl.BlockSpec((_BLOCK_ROWS, _COLS), lambda i: (i, 0))
    return pl.pallas_call(
        _ece_body,
        grid=(_GRID,),
        in_specs=[blk, blk, blk],
        out_specs=pl.BlockSpec(memory_space=pltpu.SMEM),
        out_shape=jax.ShapeDtypeStruct((24,), jnp.float32),
        interpret=interpret,
    )(logits, mask, targets)


_NW = 32
_ELEMS = _ROWS * _COLS
_PER_TILE = _ELEMS // _NW
_CHUNK = 16384
_NPIECE = _PER_TILE // _CHUNK
_NVEC = _CHUNK // 16


def _sc_hist_call(logits_flat, maskw, targets_flat):
    mesh = plsc.VectorSubcoreMesh(core_axis_name="c", subcore_axis_name="s")

    @functools.partial(
        pl.kernel,
        out_type=jax.ShapeDtypeStruct((_NW, 768), jnp.float32),
        mesh=mesh,
        compiler_params=pltpu.CompilerParams(needs_layout_passes=False),
        scratch_types=[
            pltpu.VMEM((_CHUNK,), jnp.float32),
            pltpu.VMEM((_CHUNK,), jnp.float32),
            pltpu.VMEM((_CHUNK // 4,), jnp.int32),
            pltpu.VMEM((768,), jnp.float32),
        ],
    )
    def sc_ece(x_hbm, mw_hbm, t_hbm, out_hbm, xbuf, tbuf, mbuf, hist):
        wid = lax.axis_index("s") * 2 + lax.axis_index("c")
        base = wid * _PER_TILE

        zeros16 = jnp.zeros((16,), jnp.float32)
        for k in range(48):
            hist[pl.ds(16 * k, 16)] = zeros16

        iota = lax.iota(jnp.int32, 16)
        lane_off = iota * 16
        word_sel = iota >> 2
        byte_shift = (iota & 3) * 8
        ones16 = jnp.ones((16,), jnp.float32)

        def piece_body(p, _):
            pbase = pl.multiple_of(base + p * _CHUNK, _CHUNK)
            wbase = pl.multiple_of(pbase // 4, _CHUNK // 4)
            pltpu.sync_copy(x_hbm.at[pl.ds(pbase, _CHUNK)], xbuf)
            pltpu.sync_copy(t_hbm.at[pl.ds(pbase, _CHUNK)], tbuf)
            pltpu.sync_copy(mw_hbm.at[pl.ds(wbase, _CHUNK // 4)], mbuf)

            def vec_body(i, _):
                x = xbuf[pl.ds(16 * i, 16)]
                t = tbuf[pl.ds(16 * i, 16)]
                w = plsc.load_gather(mbuf, [4 * i + word_sel])
                m = (w >> byte_shift) & 1
                mb = m == 1
                conf = 1.0 / (1.0 + jnp.exp(-jnp.abs(x)))
                acc = jnp.where(x > 0.0, t, 1.0 - t)
                b = (conf * 15.0).astype(jnp.int32)
                idx = lane_off + b
                plsc.addupdate_scatter(hist, [idx], ones16, mask=mb)
                plsc.addupdate_scatter(hist, [idx + 256], conf, mask=mb)
                plsc.addupdate_scatter(hist, [idx + 512], acc, mask=mb)
                return 0

            lax.fori_loop(0, _NVEC, vec_body, 0)
            return 0

        lax.fori_loop(0, _NPIECE, piece_body, 0)
        pltpu.sync_copy(hist, out_hbm.at[wid])

    return sc_ece(logits_flat, maskw, targets_flat)


def _sc_kernel(logits, mask, targets):
    n = logits.size
    maskw = lax.bitcast_convert_type(
        mask.astype(jnp.uint8).reshape(n // 4, 4), jnp.int32)
    part = _sc_hist_call(logits.reshape(n), maskw, targets.reshape(n))
    sums = part.reshape(_NW, 3, 16, 16).sum(axis=(0, 2))
    count = sums[0]
    sum_conf = sums[1]
    sum_acc = sums[2]
    count = count.at[14].add(count[15])[:15]
    sum_conf = sum_conf.at[14].add(sum_conf[15])[:15]
    sum_acc = sum_acc.at[14].add(sum_acc[15])[:15]
    total = jnp.float32(n)
    denom = jnp.maximum(count, 1.0)
    contrib = jnp.where(
        count > 0.0,
        jnp.abs(sum_conf / denom - sum_acc / denom) * (count / total),
        0.0,
    )
    return jnp.sum(contrib, keepdims=True)


def kernel(logits, mask, targets):
    return _sc_kernel(logits, mask, targets)


def _tc_kernel(logits, mask, targets):
    part = _partial_sums(logits, mask, targets)
    cum = part.reshape(8, 3)
    zero = jnp.zeros((1, 3), jnp.float32)
    per_bin = cum - jnp.concatenate([cum[1:], zero], axis=0)
    count = per_bin[:, 0]
    sum_conf = per_bin[:, 1]
    sum_acc = per_bin[:, 2]
    total = jnp.float32(logits.size)
    denom = jnp.maximum(count, 1.0)
    contrib = jnp.where(
        count > 0.0,
        jnp.abs(sum_conf / denom - sum_acc / denom) * (count / total),
        0.0,
    )
    return jnp.sum(contrib, keepdims=True)


# speedup vs baseline: 1.0300x; 1.0300x over previous
import functools

import jax
import jax.numpy as jnp
from jax import lax
from jax.experimental import pallas as pl
from jax.experimental.pallas import tpu as pltpu
from jax.experimental.pallas import tpu_sc as plsc

_THRESH = (0.5333333611488342, 0.6000000238418579, 0.6666666865348816,
           0.7333333492279053, 0.8000000715255737, 0.8666667342185974,
           0.9333333969116211)

_ROWS = 8192
_COLS = 2048
_BLOCK_ROWS = 256
_GRID = _ROWS // _BLOCK_ROWS


def _ece_body(x_ref, m_ref, t_ref, out_ref):
    @pl.when(pl.program_id(0) == 0)
    def _init():
        for k in range(24):
            out_ref[k] = 0.0

    x = x_ref[...]
    mf = m_ref[...].astype(jnp.float32)
    t = t_ref[...]
    conf = 0.5 * jnp.tanh(0.5 * jnp.abs(x)) + 0.5
    acc = jnp.where(x > 0, t, 1.0 - t) * mf
    confm = conf * mf
    out_ref[0] += jnp.sum(mf)
    out_ref[1] += jnp.sum(confm)
    out_ref[2] += jnp.sum(acc)
    for k, th in enumerate(_THRESH):
        g = conf > th
        base = 3 * (k + 1)
        out_ref[base + 0] += jnp.sum(jnp.where(g, mf, 0.0))
        out_ref[base + 1] += jnp.sum(jnp.where(g, confm, 0.0))
        out_ref[base + 2] += jnp.sum(jnp.where(g, acc, 0.0))


def _partial_sums(logits, mask, targets, interpret=False):
    blk = pl.BlockSpec((_BLOCK_ROWS, _COLS), lambda i: (i, 0))
    return pl.pallas_call(
        _ece_body,
        grid=(_GRID,),
        in_specs=[blk, blk, blk],
        out_specs=pl.BlockSpec(memory_space=pltpu.SMEM),
        out_shape=jax.ShapeDtypeStruct((24,), jnp.float32),
        interpret=interpret,
    )(logits, mask, targets)


_NW = 32
_ELEMS = _ROWS * _COLS
_PER_TILE = _ELEMS // _NW
_CHUNK = 16384
_NPIECE = _PER_TILE // _CHUNK
_NVEC = _CHUNK // 16
_UNROLL = 8


def _sc_hist_call(logits_flat, maskw, targets_flat):
    mesh = plsc.VectorSubcoreMesh(core_axis_name="c", subcore_axis_name="s")

    @functools.partial(
        pl.kernel,
        out_type=jax.ShapeDtypeStruct((_NW, 768), jnp.float32),
        mesh=mesh,
        compiler_params=pltpu.CompilerParams(needs_layout_passes=False),
        scratch_types=[
            pltpu.VMEM((2 * _CHUNK,), jnp.float32),
            pltpu.VMEM((2 * _CHUNK,), jnp.float32),
            pltpu.VMEM((2 * _CHUNK // 4,), jnp.int32),
            pltpu.VMEM((768,), jnp.float32),
            pltpu.SemaphoreType.DMA((2,)),
            pltpu.SemaphoreType.DMA((2,)),
            pltpu.SemaphoreType.DMA((2,)),
        ],
    )
    def sc_ece(x_hbm, mw_hbm, t_hbm, out_hbm, xbuf, tbuf, mbuf, hist,
               xsem, tsem, msem):
        wid = lax.axis_index("s") * 2 + lax.axis_index("c")
        base = wid * _PER_TILE

        zeros16 = jnp.zeros((16,), jnp.float32)
        for k in range(48):
            hist[pl.ds(16 * k, 16)] = zeros16

        iota = lax.iota(jnp.int32, 16)
        lane_off = iota * 16
        word_sel = iota >> 2
        byte_shift = (iota & 3) * 8
        ones16 = jnp.ones((16,), jnp.float32)

        def slot_refs(p):
            slot = p & 1
            sbase = pl.multiple_of(slot * _CHUNK, _CHUNK)
            swbase = pl.multiple_of(slot * (_CHUNK // 4), _CHUNK // 4)
            return (xbuf.at[pl.ds(sbase, _CHUNK)],
                    tbuf.at[pl.ds(sbase, _CHUNK)],
                    mbuf.at[pl.ds(swbase, _CHUNK // 4)], slot)

        def start_piece(p):
            xb, tb, mb, slot = slot_refs(p)
            pbase = pl.multiple_of(base + p * _CHUNK, _CHUNK)
            wbase = pl.multiple_of(pbase // 4, _CHUNK // 4)
            pltpu.async_copy(x_hbm.at[pl.ds(pbase, _CHUNK)], xb, xsem.at[slot])
            pltpu.async_copy(t_hbm.at[pl.ds(pbase, _CHUNK)], tb, tsem.at[slot])
            pltpu.async_copy(mw_hbm.at[pl.ds(wbase, _CHUNK // 4)], mb,
                             msem.at[slot])

        def wait_piece(p):
            xb, tb, mb, slot = slot_refs(p)
            pltpu.make_async_copy(x_hbm.at[pl.ds(0, _CHUNK)], xb,
                                  xsem.at[slot]).wait()
            pltpu.make_async_copy(t_hbm.at[pl.ds(0, _CHUNK)], tb,
                                  tsem.at[slot]).wait()
            pltpu.make_async_copy(mw_hbm.at[pl.ds(0, _CHUNK // 4)], mb,
                                  msem.at[slot]).wait()

        start_piece(0)

        def piece_body(p, _):
            wait_piece(p)

            @pl.when(p + 1 < _NPIECE)
            def _prefetch():
                start_piece(p + 1)

            xb, tb, mb_ref, slot = slot_refs(p)

            def vec_body(v, _):
                for u in range(_UNROLL):
                    i = _UNROLL * v + u
                    off = pl.multiple_of(16 * i, 16)
                    x = xb[pl.ds(off, 16)]
                    t = tb[pl.ds(off, 16)]
                    w = plsc.load_gather(mb_ref, [4 * i + word_sel])
                    m = (w >> byte_shift) & 1
                    mb = m == 1
                    conf = 1.0 / (1.0 + jnp.exp(-jnp.abs(x)))
                    acc = jnp.where(x > 0.0, t, 1.0 - t)
                    b = (conf * 15.0).astype(jnp.int32)
                    idx = lane_off + b
                    plsc.addupdate_scatter(hist, [idx], ones16, mask=mb)
                    plsc.addupdate_scatter(hist, [idx + 256], conf, mask=mb)
                    plsc.addupdate_scatter(hist, [idx + 512], acc, mask=mb)
                return 0

            lax.fori_loop(0, _NVEC // _UNROLL, vec_body, 0)
            return 0

        lax.fori_loop(0, _NPIECE, piece_body, 0)
        pltpu.sync_copy(hist, out_hbm.at[wid])

    return sc_ece(logits_flat, maskw, targets_flat)


def _sc_kernel(logits, mask, targets):
    n = logits.size
    maskw = lax.bitcast_convert_type(
        mask.view(jnp.uint8).reshape(n // 4, 4), jnp.int32)
    part = _sc_hist_call(logits.reshape(n), maskw, targets.reshape(n))
    sums = part.reshape(_NW, 3, 16, 16).sum(axis=(0, 2))
    count = sums[0]
    sum_conf = sums[1]
    sum_acc = sums[2]
    count = count.at[14].add(count[15])[:15]
    sum_conf = sum_conf.at[14].add(sum_conf[15])[:15]
    sum_acc = sum_acc.at[14].add(sum_acc[15])[:15]
    total = jnp.float32(n)
    denom = jnp.maximum(count, 1.0)
    contrib = jnp.where(
        count > 0.0,
        jnp.abs(sum_conf / denom - sum_acc / denom) * (count / total),
        0.0,
    )
    return jnp.sum(contrib, keepdims=True)


def kernel(logits, mask, targets):
    return _sc_kernel(logits, mask, targets)


def _tc_kernel(logits, mask, targets):
    part = _partial_sums(logits, mask, targets)
    cum = part.reshape(8, 3)
    zero = jnp.zeros((1, 3), jnp.float32)
    per_bin = cum - jnp.concatenate([cum[1:], zero], axis=0)
    count = per_bin[:, 0]
    sum_conf = per_bin[:, 1]
    sum_acc = per_bin[:, 2]
    total = jnp.float32(logits.size)
    denom = jnp.maximum(count, 1.0)
    contrib = jnp.where(
        count > 0.0,
        jnp.abs(sum_conf / denom - sum_acc / denom) * (count / total),
        0.0,
    )
    return jnp.sum(contrib, keepdims=True)
